# initial kernel scaffold (unmeasured)
import jax
import jax.numpy as jnp
from jax import lax
from jax.experimental import pallas as pl
from jax.experimental.pallas import tpu as pltpu

T_CHUNK = 256


def kernel(x, W):
    t, d = x.shape
    _, v = W.shape
    n_steps = t // T_CHUNK

    logits = jnp.dot(x, W, preferred_element_type=jnp.float32)

    def body(logits_ref, out_ref, recv_ref, send_sem, recv_sem, credit_sem):
        i = pl.program_id(0)
        my_x = lax.axis_index("x")
        my_y = lax.axis_index("y")
        my_z = lax.axis_index("z")
        partner = (1 - my_x, my_y, my_z)

        @pl.when(i == 0)
        def _():
            barrier = pltpu.get_barrier_semaphore()
            pl.semaphore_signal(
                barrier, inc=1, device_id=partner,
                device_id_type=pl.DeviceIdType.MESH,
            )
            pl.semaphore_wait(barrier, 1)

        rdma = pltpu.make_async_remote_copy(
            src_ref=logits_ref,
            dst_ref=recv_ref,
            send_sem=send_sem,
            recv_sem=recv_sem,
            device_id=partner,
            device_id_type=pl.DeviceIdType.MESH,
        )
        rdma.start()
        rdma.wait()

        pl.semaphore_signal(
            credit_sem, inc=1, device_id=partner,
            device_id_type=pl.DeviceIdType.MESH,
        )

        loc = logits_ref[...]
        rem = recv_ref[...]
        m = jnp.maximum(
            jnp.max(loc, axis=-1, keepdims=True),
            jnp.max(rem, axis=-1, keepdims=True),
        )
        el = jnp.exp(loc - m)
        er = jnp.exp(rem - m)
        denom = (
            jnp.sum(el, axis=-1, keepdims=True)
            + jnp.sum(er, axis=-1, keepdims=True)
        )
        out_ref[:, pl.ds(my_x * v, v)] = el / denom
        out_ref[:, pl.ds((1 - my_x) * v, v)] = er / denom

        pl.semaphore_wait(credit_sem, 1)

    return pl.pallas_call(
        body,
        grid=(n_steps,),
        in_specs=[pl.BlockSpec((T_CHUNK, v), lambda i: (i, 0))],
        out_specs=pl.BlockSpec((T_CHUNK, 2 * v), lambda i: (i, 0)),
        out_shape=jax.ShapeDtypeStruct((t, 2 * v), jnp.float32),
        scratch_shapes=[
            pltpu.VMEM((T_CHUNK, v), jnp.float32),
            pltpu.SemaphoreType.DMA,
            pltpu.SemaphoreType.DMA,
            pltpu.SemaphoreType.REGULAR,
        ],
        compiler_params=pltpu.CompilerParams(collective_id=0),
    )(logits)


# baseline (device time: 1075653 ns/iter reference)
import jax
import jax.numpy as jnp
from jax import lax
from jax.experimental import pallas as pl
from jax.experimental.pallas import tpu as pltpu

T_CHUNK = 128


def kernel(x, W):
    t, d = x.shape
    _, v = W.shape
    n_steps = t // T_CHUNK

    logits = jnp.dot(x, W, preferred_element_type=jnp.float32)

    def body(logits_ref, out_ref, recv_ref, send_sem, recv_sem, credit_sem):
        i = pl.program_id(0)
        my_x = lax.axis_index("x")
        my_y = lax.axis_index("y")
        my_z = lax.axis_index("z")
        partner = (1 - my_x, my_y, my_z)

        @pl.when(i == 0)
        def _():
            barrier = pltpu.get_barrier_semaphore()
            pl.semaphore_signal(
                barrier, inc=1, device_id=partner,
                device_id_type=pl.DeviceIdType.MESH,
            )
            pl.semaphore_wait(barrier, 1)

        rdma = pltpu.make_async_remote_copy(
            src_ref=logits_ref,
            dst_ref=recv_ref,
            send_sem=send_sem,
            recv_sem=recv_sem,
            device_id=partner,
            device_id_type=pl.DeviceIdType.MESH,
        )
        rdma.start()
        rdma.wait()

        pl.semaphore_signal(
            credit_sem, inc=1, device_id=partner,
            device_id_type=pl.DeviceIdType.MESH,
        )

        loc = logits_ref[...]
        rem = recv_ref[...]
        m = jnp.maximum(
            jnp.max(loc, axis=-1, keepdims=True),
            jnp.max(rem, axis=-1, keepdims=True),
        )
        el = jnp.exp(loc - m)
        er = jnp.exp(rem - m)
        denom = (
            jnp.sum(el, axis=-1, keepdims=True)
            + jnp.sum(er, axis=-1, keepdims=True)
        )
        out_ref[:, pl.ds(my_x * v, v)] = el / denom
        out_ref[:, pl.ds((1 - my_x) * v, v)] = er / denom

        pl.semaphore_wait(credit_sem, 1)

    return pl.pallas_call(
        body,
        grid=(n_steps,),
        in_specs=[pl.BlockSpec((T_CHUNK, v), lambda i: (i, 0))],
        out_specs=pl.BlockSpec((T_CHUNK, 2 * v), lambda i: (i, 0)),
        out_shape=jax.ShapeDtypeStruct((t, 2 * v), jnp.float32),
        scratch_shapes=[
            pltpu.VMEM((T_CHUNK, v), jnp.float32),
            pltpu.SemaphoreType.DMA,
            pltpu.SemaphoreType.DMA,
            pltpu.SemaphoreType.REGULAR,
        ],
        compiler_params=pltpu.CompilerParams(
            collective_id=0, vmem_limit_bytes=60 * 1024 * 1024
        ),
    )(logits)


# device time: 1012335 ns/iter; 1.0625x vs baseline; 1.0625x over previous
import jax
import jax.numpy as jnp
from jax import lax
from jax.experimental import pallas as pl
from jax.experimental.pallas import tpu as pltpu

T_CHUNK = 128
N_SLOTS = 3


def kernel(x, W):
    t, d = x.shape
    _, v = W.shape
    n = t // T_CHUNK

    logits = jnp.dot(x, W, preferred_element_type=jnp.float32)

    def body(
        logits_ref, out_ref,
        loc, recv, osl,
        load_sems, send_sems, recv_sems, out_sems, credit_sem,
    ):
        my_x = lax.axis_index("x")
        my_y = lax.axis_index("y")
        my_z = lax.axis_index("z")
        partner = (1 - my_x, my_y, my_z)
        T = T_CHUNK

        def load(j):
            return pltpu.make_async_copy(
                logits_ref.at[pl.ds(j * T, T), :],
                loc.at[j % N_SLOTS],
                load_sems.at[j % N_SLOTS],
            )

        def rdma(j):
            return pltpu.make_async_remote_copy(
                src_ref=loc.at[j % N_SLOTS],
                dst_ref=recv.at[j % N_SLOTS],
                send_sem=send_sems.at[j % N_SLOTS],
                recv_sem=recv_sems.at[j % N_SLOTS],
                device_id=partner,
                device_id_type=pl.DeviceIdType.MESH,
            )

        def out_dma(j):
            return pltpu.make_async_copy(
                osl.at[j % 2],
                out_ref.at[pl.ds(j * T, T), :],
                out_sems.at[j % 2],
            )

        load(0).start()
        barrier = pltpu.get_barrier_semaphore()
        pl.semaphore_signal(
            barrier, inc=1, device_id=partner,
            device_id_type=pl.DeviceIdType.MESH,
        )
        pl.semaphore_wait(barrier, 1)
        load(0).wait()
        rdma(0).start()
        load(1).start()

        for i in range(n):
            if i + 1 < n:
                load(i + 1).wait()
                if i + 1 >= N_SLOTS:
                    pl.semaphore_wait(credit_sem, 1)
                rdma(i + 1).start()
            if i + 2 < n:
                if i + 2 >= N_SLOTS:
                    rdma(i + 2 - N_SLOTS).wait_send()
                load(i + 2).start()

            rdma(i).wait_recv()
            if i >= 2:
                out_dma(i - 2).wait()
            lo = loc[i % N_SLOTS]
            rm = recv[i % N_SLOTS]
            m = jnp.maximum(
                jnp.max(lo, axis=-1, keepdims=True),
                jnp.max(rm, axis=-1, keepdims=True),
            )
            el = jnp.exp(lo - m)
            er = jnp.exp(rm - m)
            den = (
                jnp.sum(el, axis=-1, keepdims=True)
                + jnp.sum(er, axis=-1, keepdims=True)
            )
            osl[i % 2, :, pl.ds(my_x * v, v)] = el / den
            osl[i % 2, :, pl.ds((1 - my_x) * v, v)] = er / den
            pl.semaphore_signal(
                credit_sem, inc=1, device_id=partner,
                device_id_type=pl.DeviceIdType.MESH,
            )
            out_dma(i).start()

        for j in range(n - N_SLOTS, n):
            rdma(j).wait_send()
        out_dma(n - 2).wait()
        out_dma(n - 1).wait()
        pl.semaphore_wait(credit_sem, N_SLOTS)

    return pl.pallas_call(
        body,
        in_specs=[pl.BlockSpec(memory_space=pl.ANY)],
        out_specs=pl.BlockSpec(memory_space=pl.ANY),
        out_shape=jax.ShapeDtypeStruct((t, 2 * v), jnp.float32),
        scratch_shapes=[
            pltpu.VMEM((N_SLOTS, T_CHUNK, v), jnp.float32),
            pltpu.VMEM((N_SLOTS, T_CHUNK, v), jnp.float32),
            pltpu.VMEM((2, T_CHUNK, 2 * v), jnp.float32),
            pltpu.SemaphoreType.DMA((N_SLOTS,)),
            pltpu.SemaphoreType.DMA((N_SLOTS,)),
            pltpu.SemaphoreType.DMA((N_SLOTS,)),
            pltpu.SemaphoreType.DMA((2,)),
            pltpu.SemaphoreType.REGULAR,
        ],
        compiler_params=pltpu.CompilerParams(
            collective_id=0, vmem_limit_bytes=60 * 1024 * 1024
        ),
    )(logits)


# device time: 858901 ns/iter; 1.2524x vs baseline; 1.1786x over previous
import jax
import jax.numpy as jnp
from jax import lax
from jax.experimental import pallas as pl
from jax.experimental.pallas import tpu as pltpu

T_CHUNK = 128
LOC_SLOTS = 3
RECV_SLOTS = 2
W_TILE = 512
EXCHANGE = True


def kernel(x, W):
    t, d = x.shape
    _, v = W.shape
    n = t // T_CHUNK
    C = v // W_TILE
    G = n * C

    def body(
        x_ref, w_ref, out_ref,
        wc, xc, loc, recv, osl,
        wsems, xsems, send_sems, recv_sems, out_sem, credit_sem,
    ):
        my_x = lax.axis_index("x")
        my_y = lax.axis_index("y")
        my_z = lax.axis_index("z")
        partner = (1 - my_x, my_y, my_z)
        T = T_CHUNK

        def wdma_start(g):
            c = lax.rem(g, C) if not isinstance(g, int) else g % C
            s = lax.rem(g, 2) if not isinstance(g, int) else g % 2
            pltpu.make_async_copy(
                w_ref.at[:, pl.ds(c * W_TILE, W_TILE)],
                wc.at[s],
                wsems.at[s],
            ).start()

        def xload(j):
            return pltpu.make_async_copy(
                x_ref.at[pl.ds(j * T, T), :], xc.at[j % 2], xsems.at[j % 2]
            )

        def rdma(j):
            return pltpu.make_async_remote_copy(
                src_ref=loc.at[j % LOC_SLOTS],
                dst_ref=recv.at[j % RECV_SLOTS],
                send_sem=send_sems.at[j % LOC_SLOTS],
                recv_sem=recv_sems.at[j % RECV_SLOTS],
                device_id=partner,
                device_id_type=pl.DeviceIdType.MESH,
            )

        def out_dma(j):
            return pltpu.make_async_copy(
                osl, out_ref.at[pl.ds(j * T, T), :], out_sem
            )

        def gemm(j):
            jj = j % LOC_SLOTS
            xs = j % 2
            base = j * C

            def tile(c, _):
                g = base + c
                s = lax.rem(g, 2)
                pltpu.make_async_copy(
                    w_ref.at[:, pl.ds(lax.rem(g, C) * W_TILE, W_TILE)],
                    wc.at[s],
                    wsems.at[s],
                ).wait()

                @pl.when(g + 2 < G)
                def _():
                    wdma_start(g + 2)

                loc[jj, :, pl.ds(c * W_TILE, W_TILE)] = jnp.dot(
                    xc[xs], wc[s], preferred_element_type=jnp.float32
                )
                return 0

            lax.fori_loop(0, C, tile, 0, unroll=False)

        xload(0).start()
        if EXCHANGE:
            barrier = pltpu.get_barrier_semaphore()
            pl.semaphore_signal(
                barrier, inc=1, device_id=partner,
                device_id_type=pl.DeviceIdType.MESH,
            )
            pl.semaphore_wait(barrier, 1)
        wdma_start(0)
        wdma_start(1)
        xload(0).wait()
        gemm(0)
        xload(1).start()
        if EXCHANGE:
            rdma(0).start()

        for i in range(n):
            if i + 1 < n:
                xload(i + 1).wait()
                if EXCHANGE and i + 1 >= LOC_SLOTS:
                    rdma(i + 1 - LOC_SLOTS).wait_send()
                gemm(i + 1)
                if i + 2 < n:
                    xload(i + 2).start()
                if EXCHANGE:
                    if i + 1 >= RECV_SLOTS:
                        pl.semaphore_wait(credit_sem, 1)
                    rdma(i + 1).start()

            if EXCHANGE:
                rdma(i).wait_recv()
            if i >= 1:
                out_dma(i - 1).wait()
            lo = loc[i % LOC_SLOTS]
            rm = recv[i % RECV_SLOTS] if EXCHANGE else loc[i % LOC_SLOTS]
            m = jnp.maximum(
                jnp.max(lo, axis=-1, keepdims=True),
                jnp.max(rm, axis=-1, keepdims=True),
            )
            el = jnp.exp(lo - m)
            er = jnp.exp(rm - m)
            den = (
                jnp.sum(el, axis=-1, keepdims=True)
                + jnp.sum(er, axis=-1, keepdims=True)
            )
            osl[:, pl.ds(my_x * v, v)] = el / den
            osl[:, pl.ds((1 - my_x) * v, v)] = er / den
            if EXCHANGE:
                pl.semaphore_signal(
                    credit_sem, inc=1, device_id=partner,
                    device_id_type=pl.DeviceIdType.MESH,
                )
            out_dma(i).start()

        if EXCHANGE:
            for j in range(n - LOC_SLOTS, n):
                rdma(j).wait_send()
        out_dma(n - 1).wait()
        if EXCHANGE:
            pl.semaphore_wait(credit_sem, RECV_SLOTS)

    return pl.pallas_call(
        body,
        in_specs=[
            pl.BlockSpec(memory_space=pl.ANY),
            pl.BlockSpec(memory_space=pl.ANY),
        ],
        out_specs=pl.BlockSpec(memory_space=pl.ANY),
        out_shape=jax.ShapeDtypeStruct((t, 2 * v), jnp.float32),
        scratch_shapes=[
            pltpu.VMEM((2, d, W_TILE), jnp.float32),
            pltpu.VMEM((2, T_CHUNK, d), jnp.float32),
            pltpu.VMEM((LOC_SLOTS, T_CHUNK, v), jnp.float32),
            pltpu.VMEM((RECV_SLOTS, T_CHUNK, v), jnp.float32),
            pltpu.VMEM((T_CHUNK, 2 * v), jnp.float32),
            pltpu.SemaphoreType.DMA((2,)),
            pltpu.SemaphoreType.DMA((2,)),
            pltpu.SemaphoreType.DMA((LOC_SLOTS,)),
            pltpu.SemaphoreType.DMA((RECV_SLOTS,)),
            pltpu.SemaphoreType.DMA,
            pltpu.SemaphoreType.REGULAR,
        ],
        compiler_params=pltpu.CompilerParams(
            collective_id=0, vmem_limit_bytes=63 * 1024 * 1024
        ),
    )(x, W)
